# baseline (speedup 1.0000x reference)
import functools

import jax
import jax.numpy as jnp
from jax import lax
from jax.experimental import pallas as pl
from jax.experimental.pallas import tpu as pltpu
from jax.experimental.pallas import tpu_sc as plsc

D_MODEL = 64
SCALE = float(D_MODEL) ** 0.5

NUM_WORKERS = 32
STEP = 128
NG = 5
NW = 2
LANES = 16


def _emb_kernel(steps_per_w, idx_hbm, table_hbm, out_hbm,
                idx_v, ilists, gbufs, wbufs, gsems, wsems):
    nc = 2
    wid = lax.axis_index("s") * nc + lax.axis_index("c")
    per_w = steps_per_w * STEP
    base = wid * per_w

    pltpu.sync_copy(idx_hbm.at[pl.ds(base, per_w)], idx_v)

    def fill_ilist(j, b):
        @plsc.parallel_loop(0, STEP // LANES, unroll=4)
        def _(c):
            ilists[b][pl.ds(c * LANES, LANES)] = (
                idx_v[pl.ds(j * STEP + c * LANES, LANES)] >> 1
            )

    def gather(j, b):
        return pltpu.make_async_copy(
            table_hbm.at[ilists[b]], gbufs[b], gsems[b]
        )

    def write(j, b):
        return pltpu.make_async_copy(
            wbufs[b], out_hbm.at[pl.ds(base + j * STEP, STEP)], wsems[b]
        )

    for b in range(NG):
        fill_ilist(b, b)
        gather(b, b).start()

    period = NG * NW
    n_rounds = steps_per_w // period

    def round_body(k, _):
        for p in range(period):
            j = k * period + p
            bg = p % NG
            bw = p % NW
            gather(j, bg).wait()
            @pl.when(j >= NW)
            def _():
                write(j - NW, bw).wait()

            def group16(c, _):
                offs = (idx_v[pl.ds(j * STEP + c * LANES, LANES)] & 1) * D_MODEL
                for l in range(LANES):
                    r = c * LANES + l
                    o = offs[l]
                    for t in range(D_MODEL // LANES):
                        wbufs[bw][r, pl.ds(t * LANES, LANES)] = (
                            gbufs[bg][r, pl.ds(o + t * LANES, LANES)] * SCALE
                        )
                return 0

            lax.fori_loop(0, STEP // LANES, group16, 0, unroll=2)

            @pl.when(j + NG < steps_per_w)
            def _():
                fill_ilist(j + NG, bg)
                gather(j + NG, bg).start()

            write(j, bw).start()
        return 0

    lax.fori_loop(0, n_rounds, round_body, 0)

    for b in range(NW):
        write(steps_per_w - NW + b, b).wait()


def kernel(x, table):
    b0, b1 = x.shape
    total = b0 * b1
    n_steps = total // STEP
    steps_per_w = n_steps // NUM_WORKERS
    assert n_steps * STEP == total and steps_per_w * NUM_WORKERS == n_steps
    assert steps_per_w % (NG * NW) == 0

    idx1d = x.reshape(total).astype(jnp.int32)
    dict_len = table.shape[0]
    tbl2 = table.reshape(dict_len // 2, 2 * D_MODEL)

    mesh = plsc.VectorSubcoreMesh(core_axis_name="c", subcore_axis_name="s")
    out = pl.kernel(
        functools.partial(_emb_kernel, steps_per_w),
        mesh=mesh,
        out_type=jax.ShapeDtypeStruct((total, D_MODEL), jnp.float32),
        scratch_types=[
            pltpu.VMEM((steps_per_w * STEP,), jnp.int32),
            [pltpu.VMEM((STEP,), jnp.int32) for _ in range(NG)],
            [pltpu.VMEM((STEP, 2 * D_MODEL), jnp.float32) for _ in range(NG)],
            [pltpu.VMEM((STEP, D_MODEL), jnp.float32) for _ in range(NW)],
            [pltpu.SemaphoreType.DMA for _ in range(NG)],
            [pltpu.SemaphoreType.DMA for _ in range(NW)],
        ],
    )(idx1d, tbl2)
    return out.reshape(b0, b1, D_MODEL)


# speedup vs baseline: 1.4037x; 1.4037x over previous
import functools

import jax
import jax.numpy as jnp
from jax import lax
from jax.experimental import pallas as pl
from jax.experimental.pallas import tpu as pltpu
from jax.experimental.pallas import tpu_sc as plsc

D_MODEL = 64
SCALE = float(D_MODEL) ** 0.5
NUM_WORKERS = 32
STEP = 128
LANES = 16


def _emb_kernel(steps_per_w, idx_hbm, table_hbm, out_hbm,
                idx_v, gbuf, wbuf, gsem, wsem):
    nc = 2
    wid = lax.axis_index("s") * nc + lax.axis_index("c")
    per_w = steps_per_w * STEP
    base = wid * per_w
    pltpu.sync_copy(idx_hbm.at[pl.ds(base, per_w)], idx_v)

    def step_body(j, _):
        def row16(c, _):
            r0 = c * LANES
            chunk = idx_v[pl.ds(j * STEP + r0, LANES)]
            for l in range(LANES):
                pltpu.make_async_copy(
                    table_hbm.at[chunk[l]], gbuf.at[r0 + l], gsem
                ).start()
            return 0

        lax.fori_loop(0, STEP // LANES, row16, 0)
        pltpu.make_async_copy(
            out_hbm.at[pl.ds(0, STEP)], gbuf, gsem
        ).wait()

        def srow(g, _):
            for t in range(D_MODEL // 16):
                sl = pl.ds(t * 16, 16)
                wbuf[g, sl] = gbuf[g, sl] * SCALE
            return 0

        lax.fori_loop(0, STEP, srow, 0)
        out_slab = out_hbm.at[pl.ds(base + j * STEP, STEP)]
        pltpu.make_async_copy(wbuf, out_slab, wsem).start()
        pltpu.make_async_copy(wbuf, out_slab, wsem).wait()
        return 0

    lax.fori_loop(0, steps_per_w, step_body, 0)


def kernel(x, table):
    b0, b1 = x.shape
    total = b0 * b1
    n_steps = total // STEP
    steps_per_w = n_steps // NUM_WORKERS
    assert n_steps * STEP == total and steps_per_w * NUM_WORKERS == n_steps
    idx1d = x.reshape(total).astype(jnp.int32)

    mesh = plsc.VectorSubcoreMesh(core_axis_name="c", subcore_axis_name="s")
    out = pl.kernel(
        functools.partial(_emb_kernel, steps_per_w),
        mesh=mesh,
        out_type=jax.ShapeDtypeStruct((total, D_MODEL), jnp.float32),
        scratch_types=[
            pltpu.VMEM((6400,), jnp.int32),
            pltpu.VMEM((STEP, D_MODEL), jnp.float32),
            pltpu.VMEM((STEP, D_MODEL), jnp.float32),
            pltpu.SemaphoreType.DMA,
            pltpu.SemaphoreType.DMA,
        ],
    )(idx1d, table)
    return out.reshape(b0, b1, D_MODEL)


# speedup vs baseline: 1.5467x; 1.1018x over previous
import functools

import jax
import jax.numpy as jnp
from jax import lax
from jax.experimental import pallas as pl
from jax.experimental.pallas import tpu as pltpu
from jax.experimental.pallas import tpu_sc as plsc

D_MODEL = 64
SCALE = float(D_MODEL) ** 0.5
NUM_WORKERS = 32
STEP = 128
NBUF = 2
LANES = 16


def _emb_kernel(steps_per_w, idx_hbm, table_hbm, out_hbm,
                idx_v, gbufs, wbufs, gsems, wsems):
    nc = 2
    wid = lax.axis_index("s") * nc + lax.axis_index("c")
    per_w = steps_per_w * STEP
    base = wid * per_w
    pltpu.sync_copy(idx_hbm.at[pl.ds(base, per_w)], idx_v)

    def gather_issue(j, b):
        def row16(c, _):
            r0 = c * LANES
            chunk = idx_v[pl.ds(j * STEP + r0, LANES)]
            for l in range(LANES):
                pltpu.make_async_copy(
                    table_hbm.at[chunk[l]], gbufs[b].at[r0 + l], gsems[b]
                ).start()
            return 0

        lax.fori_loop(0, STEP // LANES, row16, 0)

    def gather_drain(b):
        pltpu.make_async_copy(
            out_hbm.at[pl.ds(0, STEP)], gbufs[b], gsems[b]
        ).wait()

    def write(j, b):
        return pltpu.make_async_copy(
            wbufs[b], out_hbm.at[pl.ds(base + j * STEP, STEP)], wsems[b]
        )

    gather_issue(0, 0)

    n_rounds = steps_per_w // NBUF

    def round_body(k, _):
        for b in range(NBUF):
            j = k * NBUF + b
            nb = (b + 1) % NBUF
            @pl.when(j + 1 < steps_per_w)
            def _():
                gather_issue(j + 1, nb)

            gather_drain(b)
            @pl.when(k > 0)
            def _():
                write(j - NBUF, b).wait()

            @plsc.parallel_loop(0, STEP, unroll=4)
            def _(i):
                for t in range(D_MODEL // LANES):
                    sl = pl.ds(t * LANES, LANES)
                    wbufs[b][i, sl] = gbufs[b][i, sl] * SCALE

            write(j, b).start()
        return 0

    lax.fori_loop(0, n_rounds, round_body, 0)

    for b in range(NBUF):
        write(steps_per_w - NBUF + b, b).wait()


def kernel(x, table):
    b0, b1 = x.shape
    total = b0 * b1
    n_steps = total // STEP
    steps_per_w = n_steps // NUM_WORKERS
    assert n_steps * STEP == total and steps_per_w * NUM_WORKERS == n_steps
    assert steps_per_w % NBUF == 0

    idx1d = x.reshape(total).astype(jnp.int32)

    mesh = plsc.VectorSubcoreMesh(core_axis_name="c", subcore_axis_name="s")
    out = pl.kernel(
        functools.partial(_emb_kernel, steps_per_w),
        mesh=mesh,
        out_type=jax.ShapeDtypeStruct((total, D_MODEL), jnp.float32),
        scratch_types=[
            pltpu.VMEM((steps_per_w * STEP,), jnp.int32),
            [pltpu.VMEM((STEP, D_MODEL), jnp.float32) for _ in range(NBUF)],
            [pltpu.VMEM((STEP, D_MODEL), jnp.float32) for _ in range(NBUF)],
            [pltpu.SemaphoreType.DMA for _ in range(NBUF)],
            [pltpu.SemaphoreType.DMA for _ in range(NBUF)],
        ],
    )(idx1d, table)
    return out.reshape(b0, b1, D_MODEL)
